# baseline (speedup 1.0000x reference)
import functools

import jax
import jax.numpy as jnp
from jax import lax
from jax.experimental import pallas as pl
from jax.experimental.pallas import tpu as pltpu
from jax.experimental.pallas import tpu_sc as plsc

_K = 64
_TW = 128


def _lut_prep_body(uy_ref, p_ref):
    uy3 = uy_ref[...]
    uy = uy3.reshape(uy3.shape[0] * uy3.shape[1], uy3.shape[2])
    kp1 = uy.shape[1]
    hgt = jnp.maximum(uy, 0.0) + jnp.log(1.0 + jnp.exp(-jnp.abs(uy)))

    row = lax.broadcasted_iota(jnp.int32, (kp1, _TW), 0)
    col = lax.broadcasted_iota(jnp.int32, (kp1, _TW), 1)
    colc = jnp.minimum(col, _K - 1)
    f32 = jnp.float32
    cum_m = (row <= colc).astype(f32)
    nxt_m = (row == colc + 1).astype(f32)
    tot_m = jnp.ones((kp1, _TW), f32)
    fst_m = (row == 0).astype(f32)

    y0 = jnp.dot(hgt, cum_m, preferred_element_type=f32)
    d = jnp.dot(hgt, nxt_m, preferred_element_type=f32)
    y_last = jnp.dot(hgt, tot_m, preferred_element_type=f32)
    y_first = jnp.dot(hgt, fst_m, preferred_element_type=f32)

    r = 1.0 / (y_last - y_first)
    d_n = d * r
    y0_n = (y0 - y_first) * r

    d_r = d_n.astype(jnp.bfloat16).astype(f32)
    j = jnp.minimum(lax.broadcasted_iota(jnp.int32, d.shape, 1),
                    _K - 1).astype(f32)
    t_r = (y0_n - j * d_r).astype(jnp.bfloat16).astype(f32)
    d_b = lax.bitcast_convert_type(d_r, jnp.int32)
    t_b = lax.bitcast_convert_type(t_r, jnp.int32)
    p_ref[...] = jnp.bitwise_or(
        jnp.bitwise_and(d_b, jnp.int32(-65536)),
        lax.shift_right_logical(t_b, 16))


def _lut_prep(uy3):
    p = uy3.shape[0] * uy3.shape[1]
    return pl.pallas_call(
        _lut_prep_body,
        out_shape=jax.ShapeDtypeStruct((p, _TW), jnp.int32),
    )(uy3)


def _make_sc_apply(bdim, cdim, h, w, num_workers, lanes):
    num_planes = bdim * cdim
    rows = h // num_workers
    chunk = rows * w
    tw = _TW

    mesh = plsc.VectorSubcoreMesh(core_axis_name="c", subcore_axis_name="s")

    @functools.partial(
        pl.kernel,
        out_type=jax.ShapeDtypeStruct((bdim, cdim, h, w), jnp.float32),
        mesh=mesh,
        compiler_params=pltpu.CompilerParams(needs_layout_passes=False),
        scratch_types=[
            pltpu.VMEM((num_planes * _TW,), jnp.int32),
            pltpu.VMEM((rows, w), jnp.float32),
            pltpu.VMEM((rows, w), jnp.float32),
            pltpu.VMEM((rows, w), jnp.float32),
            pltpu.VMEM((rows, w), jnp.float32),
            pltpu.SemaphoreType.DMA,
            pltpu.SemaphoreType.DMA,
            pltpu.SemaphoreType.DMA,
            pltpu.SemaphoreType.DMA,
        ],
    )
    def sc_apply(x_hbm, p_hbm, out_hbm, p_v, xb0, xb1, ob0, ob1,
                 si0, si1, so0, so1):
        ncores = jax.lax.axis_size("c")
        wid = lax.axis_index("s") * ncores + lax.axis_index("c")
        row0 = wid * rows
        in_sems = (si0, si1)
        out_sems = (so0, so1)
        xbufs = (xb0, xb1)
        obufs = (ob0, ob1)

        def in_copy(p, b):
            return pltpu.make_async_copy(
                x_hbm.at[p // cdim, p % cdim, pl.ds(row0, rows), :],
                xbufs[b], in_sems[b])

        def out_copy(p, b):
            return pltpu.make_async_copy(
                obufs[b], out_hbm.at[p // cdim, p % cdim, pl.ds(row0, rows), :],
                out_sems[b])

        wshift = w.bit_length() - 1
        assert (1 << wshift) == w

        def compute(p, b):
            xref = xbufs[b]
            oref = obufs[b]
            p_p = p_v.at[pl.ds(p * tw, tw)]

            @plsc.parallel_loop(0, chunk, step=lanes, unroll=16)
            def body(i):
                r = lax.shift_right_logical(i, wshift)
                col = jnp.bitwise_and(i, w - 1)
                col = pl.multiple_of(col, lanes)
                xv = xref[r, pl.ds(col, lanes)]
                xk = xv * jnp.float32(_K)
                xi = xk.astype(jnp.int32)
                wv = plsc.load_gather(p_p, [xi])
                dy = plsc.bitcast(
                    jnp.bitwise_and(wv, jnp.int32(-65536)), jnp.float32)
                tv = plsc.bitcast(lax.shift_left(wv, 16), jnp.float32)
                oref[r, pl.ds(col, lanes)] = dy * xk + tv

        in_copy(0, 0).start()
        pltpu.sync_copy(p_hbm, p_v)
        for p in range(num_planes):
            b = p & 1
            if p + 1 < num_planes:
                in_copy(p + 1, 1 - b).start()
            in_copy(p, b).wait()
            if p >= 2:
                out_copy(p - 2, b).wait()
            compute(p, b)
            out_copy(p, b).start()
        out_copy(num_planes - 2, 0).wait()
        out_copy(num_planes - 1, 1).wait()

    return sc_apply


def kernel(x, un_normalized_y):
    b, c, h, w = x.shape
    num_workers = 32
    lanes = 16
    assert h % num_workers == 0 and w % lanes == 0

    packed = _lut_prep(un_normalized_y)

    apply_fn = _make_sc_apply(b, c, h, w, num_workers, lanes)
    return apply_fn(x, packed.reshape(-1))


# speedup vs baseline: 1.0602x; 1.0602x over previous
import functools

import jax
import jax.numpy as jnp
from jax import lax
from jax.experimental import pallas as pl
from jax.experimental.pallas import tpu as pltpu
from jax.experimental.pallas import tpu_sc as plsc

_K = 64
_TW = 128


def _lut_prep_body(uy_ref, p_ref):
    uy3 = uy_ref[...]
    uy = uy3.reshape(uy3.shape[0] * uy3.shape[1], uy3.shape[2])
    kp1 = uy.shape[1]
    hgt = jnp.maximum(uy, 0.0) + jnp.log(1.0 + jnp.exp(-jnp.abs(uy)))

    row = lax.broadcasted_iota(jnp.int32, (kp1, _TW), 0)
    col = lax.broadcasted_iota(jnp.int32, (kp1, _TW), 1)
    colc = jnp.minimum(col, _K - 1)
    f32 = jnp.float32
    cum_m = (row <= colc).astype(f32)
    nxt_m = (row == colc + 1).astype(f32)
    tot_m = jnp.ones((kp1, _TW), f32)
    fst_m = (row == 0).astype(f32)

    y0 = jnp.dot(hgt, cum_m, preferred_element_type=f32)
    d = jnp.dot(hgt, nxt_m, preferred_element_type=f32)
    y_last = jnp.dot(hgt, tot_m, preferred_element_type=f32)
    y_first = jnp.dot(hgt, fst_m, preferred_element_type=f32)

    r = 1.0 / (y_last - y_first)
    d_n = d * r
    y0_n = (y0 - y_first) * r

    d_r = d_n.astype(jnp.bfloat16).astype(f32)
    j = jnp.minimum(lax.broadcasted_iota(jnp.int32, d.shape, 1),
                    _K - 1).astype(f32)
    d_b = jnp.bitwise_and(lax.bitcast_convert_type(d_r, jnp.int32),
                          jnp.int32(-65536))
    dy_eff = d_r
    for _ in range(4):
        t_r = (y0_n - j * dy_eff).astype(jnp.bfloat16).astype(f32)
        t_b = lax.shift_right_logical(
            lax.bitcast_convert_type(t_r, jnp.int32), 16)
        word = jnp.bitwise_or(d_b, t_b)
        dy_eff = lax.bitcast_convert_type(word, f32)
    p_ref[...] = word


def _lut_prep(uy3):
    p = uy3.shape[0] * uy3.shape[1]
    return pl.pallas_call(
        _lut_prep_body,
        out_shape=jax.ShapeDtypeStruct((p, _TW), jnp.int32),
    )(uy3)


def _make_sc_apply(bdim, cdim, h, w, num_workers, lanes):
    num_planes = bdim * cdim
    rows = h // num_workers
    chunk = rows * w
    tw = _TW

    mesh = plsc.VectorSubcoreMesh(core_axis_name="c", subcore_axis_name="s")

    @functools.partial(
        pl.kernel,
        out_type=jax.ShapeDtypeStruct((bdim, cdim, h, w), jnp.float32),
        mesh=mesh,
        compiler_params=pltpu.CompilerParams(needs_layout_passes=False),
        scratch_types=[
            pltpu.VMEM((num_planes * _TW,), jnp.int32),
            pltpu.VMEM((rows, w), jnp.float32),
            pltpu.VMEM((rows, w), jnp.float32),
            pltpu.VMEM((rows, w), jnp.float32),
            pltpu.VMEM((rows, w), jnp.float32),
            pltpu.SemaphoreType.DMA,
            pltpu.SemaphoreType.DMA,
            pltpu.SemaphoreType.DMA,
            pltpu.SemaphoreType.DMA,
        ],
    )
    def sc_apply(x_hbm, p_hbm, out_hbm, p_v, xb0, xb1, ob0, ob1,
                 si0, si1, so0, so1):
        ncores = jax.lax.axis_size("c")
        wid = lax.axis_index("s") * ncores + lax.axis_index("c")
        row0 = wid * rows
        in_sems = (si0, si1)
        out_sems = (so0, so1)
        xbufs = (xb0, xb1)
        obufs = (ob0, ob1)

        def in_copy(p, b):
            return pltpu.make_async_copy(
                x_hbm.at[p // cdim, p % cdim, pl.ds(row0, rows), :],
                xbufs[b], in_sems[b])

        def out_copy(p, b):
            return pltpu.make_async_copy(
                obufs[b], out_hbm.at[p // cdim, p % cdim, pl.ds(row0, rows), :],
                out_sems[b])

        wshift = w.bit_length() - 1
        assert (1 << wshift) == w

        def compute(p, b):
            xref = xbufs[b]
            oref = obufs[b]
            p_p = p_v.at[pl.ds(p * tw, tw)]

            @plsc.parallel_loop(0, chunk, step=lanes, unroll=8)
            def body(i):
                r = lax.shift_right_logical(i, wshift)
                col = jnp.bitwise_and(i, w - 1)
                col = pl.multiple_of(col, lanes)
                xv = xref[r, pl.ds(col, lanes)]
                xk = xv * jnp.float32(_K)
                xi = xk.astype(jnp.int32)
                wv = plsc.load_gather(p_p, [xi])
                dy = plsc.bitcast(wv, jnp.float32)
                tv = plsc.bitcast(lax.shift_left(wv, 16), jnp.float32)
                oref[r, pl.ds(col, lanes)] = dy * xk + tv

        in_copy(0, 0).start()
        pltpu.sync_copy(p_hbm, p_v)
        for p in range(num_planes):
            b = p & 1
            if p + 1 < num_planes:
                in_copy(p + 1, 1 - b).start()
            in_copy(p, b).wait()
            if p >= 2:
                out_copy(p - 2, b).wait()
            compute(p, b)
            out_copy(p, b).start()
        out_copy(num_planes - 2, 0).wait()
        out_copy(num_planes - 1, 1).wait()

    return sc_apply


def kernel(x, un_normalized_y):
    b, c, h, w = x.shape
    num_workers = 32
    lanes = 16
    assert h % num_workers == 0 and w % lanes == 0

    packed = _lut_prep(un_normalized_y)

    apply_fn = _make_sc_apply(b, c, h, w, num_workers, lanes)
    return apply_fn(x, packed.reshape(-1))


# speedup vs baseline: 1.2191x; 1.1499x over previous
import functools

import jax
import jax.numpy as jnp
from jax import lax
from jax.experimental import pallas as pl
from jax.experimental.pallas import tpu as pltpu
from jax.experimental.pallas import tpu_sc as plsc

_K = 64
_TW = 128


def _lut_prep_body(uy_ref, p_ref):
    uy3 = uy_ref[...]
    uy = uy3.reshape(uy3.shape[0] * uy3.shape[1], uy3.shape[2])
    kp1 = uy.shape[1]
    hgt = jnp.maximum(uy, 0.0) + jnp.log(1.0 + jnp.exp(-jnp.abs(uy)))

    row = lax.broadcasted_iota(jnp.int32, (kp1, _TW), 0)
    col = lax.broadcasted_iota(jnp.int32, (kp1, _TW), 1)
    colc = jnp.minimum(col, _K - 1)
    f32 = jnp.float32
    cum_m = (row <= colc).astype(f32)
    nxt_m = (row == colc + 1).astype(f32)
    tot_m = jnp.ones((kp1, _TW), f32)
    fst_m = (row == 0).astype(f32)

    y0 = jnp.dot(hgt, cum_m, preferred_element_type=f32)
    d = jnp.dot(hgt, nxt_m, preferred_element_type=f32)
    y_last = jnp.dot(hgt, tot_m, preferred_element_type=f32)
    y_first = jnp.dot(hgt, fst_m, preferred_element_type=f32)

    r = 1.0 / (y_last - y_first)
    d_n = d * r
    y0_n = (y0 - y_first) * r

    d_r = d_n.astype(jnp.bfloat16).astype(f32)
    j = jnp.minimum(lax.broadcasted_iota(jnp.int32, d.shape, 1),
                    _K - 1).astype(f32)
    d_b = jnp.bitwise_and(lax.bitcast_convert_type(d_r, jnp.int32),
                          jnp.int32(-65536))
    dy_eff = d_r
    for _ in range(4):
        t_r = (y0_n - j * dy_eff).astype(jnp.bfloat16).astype(f32)
        t_b = lax.shift_right_logical(
            lax.bitcast_convert_type(t_r, jnp.int32), 16)
        word = jnp.bitwise_or(d_b, t_b)
        dy_eff = lax.bitcast_convert_type(word, f32)
    p_ref[...] = word


def _lut_prep(uy3):
    p = uy3.shape[0] * uy3.shape[1]
    return pl.pallas_call(
        _lut_prep_body,
        out_shape=jax.ShapeDtypeStruct((p, _TW), jnp.int32),
    )(uy3)


def _make_sc_apply(bdim, cdim, h, w, num_workers, lanes):
    num_planes = bdim * cdim
    rows = h // num_workers
    chunk = rows * w
    tw = _TW

    mesh = plsc.VectorSubcoreMesh(core_axis_name="c", subcore_axis_name="s")

    @functools.partial(
        pl.kernel,
        out_type=jax.ShapeDtypeStruct((bdim, cdim, h, w), jnp.float32),
        mesh=mesh,
        compiler_params=pltpu.CompilerParams(needs_layout_passes=False),
        scratch_types=[
            pltpu.VMEM((num_planes * _TW,), jnp.int32),
            pltpu.VMEM((rows, w), jnp.float32),
            pltpu.VMEM((rows, w), jnp.float32),
            pltpu.VMEM((rows, w), jnp.float32),
            pltpu.VMEM((rows, w), jnp.float32),
            pltpu.VMEM((rows, w), jnp.float32),
            pltpu.SemaphoreType.DMA,
            pltpu.SemaphoreType.DMA,
            pltpu.SemaphoreType.DMA,
            pltpu.SemaphoreType.DMA,
            pltpu.SemaphoreType.DMA,
        ],
    )
    def sc_apply(x_hbm, p_hbm, out_hbm, p_v, xb0, xb1, xb2, ob0, ob1,
                 si0, si1, si2, so0, so1):
        ncores = jax.lax.axis_size("c")
        wid = lax.axis_index("s") * ncores + lax.axis_index("c")
        row0 = wid * rows
        in_sems = (si0, si1, si2)
        out_sems = (so0, so1)
        xbufs = (xb0, xb1, xb2)
        obufs = (ob0, ob1)

        def in_copy(p, b):
            return pltpu.make_async_copy(
                x_hbm.at[p // cdim, p % cdim, pl.ds(row0, rows), :],
                xbufs[b], in_sems[b])

        def out_copy(p, b):
            return pltpu.make_async_copy(
                obufs[b], out_hbm.at[p // cdim, p % cdim, pl.ds(row0, rows), :],
                out_sems[b])

        wshift = w.bit_length() - 1
        assert (1 << wshift) == w

        def compute(p, bi, bo):
            xref = xbufs[bi]
            oref = obufs[bo]
            p_p = p_v.at[pl.ds(p * tw, tw)]

            @plsc.parallel_loop(0, chunk, step=lanes, unroll=8)
            def body(i):
                r = lax.shift_right_logical(i, wshift)
                col = jnp.bitwise_and(i, w - 1)
                col = pl.multiple_of(col, lanes)
                xv = xref[r, pl.ds(col, lanes)]
                xk = xv * jnp.float32(_K)
                xi = xk.astype(jnp.int32)
                wv = plsc.load_gather(p_p, [xi])
                dy = plsc.bitcast(wv, jnp.float32)
                tv = plsc.bitcast(lax.shift_left(wv, 16), jnp.float32)
                oref[r, pl.ds(col, lanes)] = dy * xk + tv

        in_copy(0, 0).start()
        in_copy(1, 1).start()
        pltpu.sync_copy(p_hbm, p_v)
        for p in range(num_planes):
            bi = p % 3
            bo = p & 1
            if p + 2 < num_planes:
                in_copy(p + 2, (p + 2) % 3).start()
            in_copy(p, bi).wait()
            if p >= 2:
                out_copy(p - 2, bo).wait()
            compute(p, bi, bo)
            out_copy(p, bo).start()
        out_copy(num_planes - 2, 0).wait()
        out_copy(num_planes - 1, 1).wait()

    return sc_apply


def kernel(x, un_normalized_y):
    b, c, h, w = x.shape
    num_workers = 32
    lanes = 16
    assert h % num_workers == 0 and w % lanes == 0

    packed = _lut_prep(un_normalized_y)

    apply_fn = _make_sc_apply(b, c, h, w, num_workers, lanes)
    return apply_fn(x, packed.reshape(-1))


# speedup vs baseline: 1.2312x; 1.0100x over previous
import functools

import jax
import jax.numpy as jnp
from jax import lax
from jax.experimental import pallas as pl
from jax.experimental.pallas import tpu as pltpu
from jax.experimental.pallas import tpu_sc as plsc

_K = 64
_TW = 128


def _lut_prep_body(uy_ref, p_ref):
    uy3 = uy_ref[...]
    uy = uy3.reshape(uy3.shape[0] * uy3.shape[1], uy3.shape[2])
    kp1 = uy.shape[1]
    hgt = jnp.maximum(uy, 0.0) + jnp.log(1.0 + jnp.exp(-jnp.abs(uy)))

    row = lax.broadcasted_iota(jnp.int32, (kp1, _TW), 0)
    col = lax.broadcasted_iota(jnp.int32, (kp1, _TW), 1)
    colc = jnp.minimum(col, _K - 1)
    f32 = jnp.float32
    cum_m = (row <= colc).astype(f32)
    nxt_m = (row == colc + 1).astype(f32)
    tot_m = jnp.ones((kp1, _TW), f32)
    fst_m = (row == 0).astype(f32)

    y0 = jnp.dot(hgt, cum_m, preferred_element_type=f32)
    d = jnp.dot(hgt, nxt_m, preferred_element_type=f32)
    y_last = jnp.dot(hgt, tot_m, preferred_element_type=f32)
    y_first = jnp.dot(hgt, fst_m, preferred_element_type=f32)

    r = 1.0 / (y_last - y_first)
    d_n = d * r
    y0_n = (y0 - y_first) * r

    d_r = d_n.astype(jnp.bfloat16).astype(f32)
    j = jnp.minimum(lax.broadcasted_iota(jnp.int32, d.shape, 1),
                    _K - 1).astype(f32)
    d_b = jnp.bitwise_and(lax.bitcast_convert_type(d_r, jnp.int32),
                          jnp.int32(-65536))
    dy_eff = d_r
    for _ in range(4):
        t_r = (y0_n - j * dy_eff).astype(jnp.bfloat16).astype(f32)
        t_b = lax.shift_right_logical(
            lax.bitcast_convert_type(t_r, jnp.int32), 16)
        word = jnp.bitwise_or(d_b, t_b)
        dy_eff = lax.bitcast_convert_type(word, f32)
    p_ref[...] = word


def _lut_prep(uy3):
    p = uy3.shape[0] * uy3.shape[1]
    return pl.pallas_call(
        _lut_prep_body,
        out_shape=jax.ShapeDtypeStruct((p, _TW), jnp.int32),
    )(uy3)


def _make_sc_apply(bdim, cdim, h, w, num_workers, lanes):
    num_planes = bdim * cdim
    rows = h // num_workers
    chunk = rows * w
    tw = _TW

    mesh = plsc.VectorSubcoreMesh(core_axis_name="c", subcore_axis_name="s")

    @functools.partial(
        pl.kernel,
        out_type=jax.ShapeDtypeStruct((bdim, cdim, h, w), jnp.float32),
        mesh=mesh,
        compiler_params=pltpu.CompilerParams(needs_layout_passes=False),
        scratch_types=[
            pltpu.VMEM((num_planes * _TW,), jnp.int32),
            pltpu.VMEM((rows, w), jnp.float32),
            pltpu.VMEM((rows, w), jnp.float32),
            pltpu.VMEM((rows, w), jnp.float32),
            pltpu.VMEM((rows, w), jnp.float32),
            pltpu.VMEM((rows, w), jnp.float32),
            pltpu.VMEM((rows, w), jnp.float32),
            pltpu.VMEM((rows, w), jnp.float32),
            pltpu.SemaphoreType.DMA,
            pltpu.SemaphoreType.DMA,
            pltpu.SemaphoreType.DMA,
            pltpu.SemaphoreType.DMA,
            pltpu.SemaphoreType.DMA,
            pltpu.SemaphoreType.DMA,
            pltpu.SemaphoreType.DMA,
        ],
    )
    def sc_apply(x_hbm, p_hbm, out_hbm, p_v, xb0, xb1, xb2, xb3,
                 ob0, ob1, ob2, si0, si1, si2, si3, so0, so1, so2):
        ncores = jax.lax.axis_size("c")
        wid = lax.axis_index("s") * ncores + lax.axis_index("c")
        row0 = wid * rows
        in_sems = (si0, si1, si2, si3)
        out_sems = (so0, so1, so2)
        xbufs = (xb0, xb1, xb2, xb3)
        obufs = (ob0, ob1, ob2)

        def in_copy(p, b):
            return pltpu.make_async_copy(
                x_hbm.at[p // cdim, p % cdim, pl.ds(row0, rows), :],
                xbufs[b], in_sems[b])

        def out_copy(p, b):
            return pltpu.make_async_copy(
                obufs[b], out_hbm.at[p // cdim, p % cdim, pl.ds(row0, rows), :],
                out_sems[b])

        wshift = w.bit_length() - 1
        assert (1 << wshift) == w

        def compute(p, bi, bo):
            xref = xbufs[bi]
            oref = obufs[bo]
            p_p = p_v.at[pl.ds(p * tw, tw)]

            @plsc.parallel_loop(0, chunk, step=lanes, unroll=8)
            def body(i):
                r = lax.shift_right_logical(i, wshift)
                col = jnp.bitwise_and(i, w - 1)
                col = pl.multiple_of(col, lanes)
                xv = xref[r, pl.ds(col, lanes)]
                xk = xv * jnp.float32(_K)
                xi = xk.astype(jnp.int32)
                wv = plsc.load_gather(p_p, [xi])
                dy = plsc.bitcast(wv, jnp.float32)
                tv = plsc.bitcast(lax.shift_left(wv, 16), jnp.float32)
                oref[r, pl.ds(col, lanes)] = dy * xk + tv

        nin = len(xbufs)
        nout = len(obufs)
        for q in range(nin - 1):
            in_copy(q, q).start()
        pltpu.sync_copy(p_hbm, p_v)
        for p in range(num_planes):
            bi = p % nin
            bo = p % nout
            if p + nin - 1 < num_planes:
                in_copy(p + nin - 1, (p + nin - 1) % nin).start()
            in_copy(p, bi).wait()
            if p >= nout:
                out_copy(p - nout, bo).wait()
            compute(p, bi, bo)
            out_copy(p, bo).start()
        for q in range(num_planes - nout, num_planes):
            out_copy(q, q % nout).wait()

    return sc_apply


def kernel(x, un_normalized_y):
    b, c, h, w = x.shape
    num_workers = 32
    lanes = 16
    assert h % num_workers == 0 and w % lanes == 0

    packed = _lut_prep(un_normalized_y)

    apply_fn = _make_sc_apply(b, c, h, w, num_workers, lanes)
    return apply_fn(x, packed.reshape(-1))
